# baseline (speedup 1.0000x reference)
import functools

import jax
import jax.numpy as jnp
from jax import lax
from jax.experimental import pallas as pl
from jax.experimental.pallas import tpu as pltpu
from jax.experimental.pallas import tpu_sc as plsc

_N = 10000
_E = 160000
_D = 256
_HD = _D // 2
_R = 4
_NC, _NS, _L = 2, 16, 16
_NW = _NC * _NS
_PW = _E // _NW
_C = 80
_NCH = -(-_PW // _C)
_TAIL = _PW - (_NCH - 1) * _C
_PAD = _NCH * _C


def _rne16(u):
    return (u + 0x7FFF + ((u >> 16) & 1)) >> 16


def _pack_pair_bf16(lo_f32, hi_f32):
    ulo = lax.bitcast_convert_type(lo_f32, jnp.int32)
    uhi = lax.bitcast_convert_type(hi_f32, jnp.int32)
    return (_rne16(ulo) & 0xFFFF) | (_rne16(uhi) << 16)


def _prep(h, w_relation, ei, et):
    bn = 1000
    rows = _E // 128 // 10

    def body(h_ref, w_ref, ei_ref, et_ref, hb_ref, wp_ref, pidx_ref):
        x = h_ref[...]
        hb_ref[...] = _pack_pair_bf16(x[:, :_HD], x[:, _HD:])
        wv = w_ref[...]
        wp_ref[...] = _pack_pair_bf16(wv[:, :_HD], wv[:, _HD:])
        pidx_ref[...] = ei_ref[0] | (ei_ref[1] << 14) | (et_ref[...] << 28)

    hb, wp, pidx = pl.pallas_call(
        body,
        grid=(10,),
        in_specs=[
            pl.BlockSpec((bn, _D), lambda i: (i, 0)),
            pl.BlockSpec((_R, _D), lambda i: (0, 0)),
            pl.BlockSpec((2, 1, rows, 128), lambda i: (0, i, 0, 0)),
            pl.BlockSpec((1, rows, 128), lambda i: (i, 0, 0)),
        ],
        out_specs=[
            pl.BlockSpec((bn, _HD), lambda i: (i, 0)),
            pl.BlockSpec((_R, _HD), lambda i: (0, 0)),
            pl.BlockSpec((1, rows, 128), lambda i: (i, 0, 0)),
        ],
        out_shape=[
            jax.ShapeDtypeStruct((_N, _HD), jnp.int32),
            jax.ShapeDtypeStruct((_R, _HD), jnp.int32),
            jax.ShapeDtypeStruct((10, rows, 128), jnp.int32),
        ],
    )(h, w_relation, ei.reshape(2, 10, rows, 128), et.reshape(10, rows, 128))
    return hb, wp.reshape(_R * _HD), pidx.reshape(_E)


_mesh = plsc.VectorSubcoreMesh(
    core_axis_name="c", subcore_axis_name="s", num_cores=_NC, num_subcores=_NS
)


@functools.partial(
    pl.kernel,
    out_type=jax.ShapeDtypeStruct((_E,), jnp.float32),
    mesh=_mesh,
    compiler_params=pltpu.CompilerParams(needs_layout_passes=False),
    scratch_types=[
        pltpu.VMEM_SHARED((_N, _HD), jnp.int32),
        pltpu.VMEM((_PAD,), jnp.int32),
        pltpu.VMEM((2, _C), jnp.int32),
        pltpu.VMEM((2, _C), jnp.int32),
        pltpu.VMEM((2, _C), jnp.int32),
        pltpu.VMEM((_R * _HD,), jnp.int32),
        pltpu.VMEM((2, _C, _HD), jnp.int32),
        pltpu.VMEM((2, _C, _HD), jnp.int32),
        pltpu.VMEM((_C,), jnp.float32),
        pltpu.SemaphoreType.DMA,
        pltpu.SemaphoreType.DMA,
        pltpu.SemaphoreType.DMA,
        pltpu.SemaphoreType.DMA,
    ],
)
def _distmult_sc(hb_hbm, wp_hbm, pidx_hbm, out_hbm,
                 tab, pidx, sidx, didx, etv, wp, s_rows, d_rows, outv,
                 sem_s0, sem_s1, sem_d0, sem_d1):
    sems = ((sem_s0, sem_d0), (sem_s1, sem_d1))
    sid = lax.axis_index("s")
    wid = sid * _NC + lax.axis_index("c")
    base = wid * _PW
    lane = lax.iota(jnp.int32, _L)
    zero16 = jnp.zeros((_L,), jnp.int32)

    toff = pl.multiple_of(sid * 624, 8)
    pltpu.sync_copy(hb_hbm.at[pl.ds(toff, 624)], tab.at[pl.ds(toff, 624)])

    @pl.when(sid == 0)
    def _tab_rem():
        pltpu.sync_copy(hb_hbm.at[pl.ds(624 * _NS, _N - 624 * _NS)],
                        tab.at[pl.ds(624 * _NS, _N - 624 * _NS)])

    pltpu.sync_copy(wp_hbm, wp)

    pltpu.sync_copy(pidx_hbm.at[pl.ds(base, _PW)], pidx.at[pl.ds(0, _PW)])
    pidx[pl.ds(_PW, _L)] = zero16
    pidx[pl.ds(_PW + _L, _L)] = zero16
    pidx[pl.ds(_PAD - _L, _L)] = zero16

    plsc.subcore_barrier()

    def _coff(j):
        return j * _C if isinstance(j, int) else pl.multiple_of(j * _C, _C)

    def _fetch(j, b):
        off = _coff(j)
        for q in range(_C // _L):
            v = pidx[pl.ds(off + q * _L, _L)]
            sidx[b, pl.ds(q * _L, _L)] = v & 0x3FFF
            didx[b, pl.ds(q * _L, _L)] = (v >> 14) & 0x3FFF
            etv[b, pl.ds(q * _L, _L)] = (v >> 28) & 0x3
        pltpu.async_copy(hb_hbm.at[sidx.at[b]], s_rows.at[b], sems[b][0])
        pltpu.async_copy(tab.at[didx.at[b]], d_rows.at[b], sems[b][1])

    def _consume(j, b, tail=False):
        pltpu.make_async_copy(hb_hbm.at[sidx.at[b]], s_rows.at[b],
                              sems[b][0]).wait()
        pltpu.make_async_copy(tab.at[didx.at[b]], d_rows.at[b],
                              sems[b][1]).wait()

        @pl.loop(0, _C // _L)
        def _group(g):
            et_vreg = etv[b, pl.ds(pl.multiple_of(g * _L, _L), _L)]
            res = jnp.zeros((_L,), jnp.float32)
            for e2 in range(_L):
                row = g * _L + e2
                et_splat = jnp.take_along_axis(
                    et_vreg, jnp.full((_L,), e2, jnp.int32), axis=0,
                    mode="promise_in_bounds")
                widx = et_splat * _HD + lane
                acc0 = jnp.zeros((_L,), jnp.float32)
                acc1 = jnp.zeros((_L,), jnp.float32)
                for k in range(_HD // _L):
                    ksl = pl.ds(k * _L, _L)
                    s32 = s_rows[b, row, ksl]
                    d32 = d_rows[b, row, ksl]
                    w32 = plsc.load_gather(wp, [widx + (k * _L)])
                    sb = plsc.bitcast(s32, jnp.bfloat16)
                    db = plsc.bitcast(d32, jnp.bfloat16)
                    wb = plsc.bitcast(w32, jnp.bfloat16)
                    p = (sb * db) * wb
                    lo, hi = plsc.unpack(p, format=plsc.PackFormat.INTERLEAVED)
                    acc0 = acc0 + lo
                    acc1 = acc1 + hi
                acc = acc0 + acc1
                for sh in (8, 4, 2, 1):
                    perm = jnp.bitwise_xor(lane, sh)
                    acc = acc + jnp.take_along_axis(
                        acc, perm, axis=0, mode="promise_in_bounds")
                res = jnp.where(lane == e2, acc, res)
            outv[pl.ds(pl.multiple_of(g * _L, _L), _L)] = res

        eoff = base + _coff(j)
        if tail:
            pltpu.sync_copy(outv.at[pl.ds(0, _TAIL)],
                            out_hbm.at[pl.ds(eoff, _TAIL)])
        else:
            pltpu.sync_copy(outv, out_hbm.at[pl.ds(eoff, _C)])

    _fetch(0, 0)

    @pl.loop(0, _NCH - 2, step=2)
    def _rounds(t):
        for b in range(2):
            _fetch(t + b + 1, 1 - b)
            _consume(t + b, b)

    _consume(_NCH - 1, 0, tail=True)


def kernel(h, edge_index, edge_type, w_relation):
    hb, wp, pidx = _prep(h, w_relation, edge_index.astype(jnp.int32),
                         edge_type.astype(jnp.int32))
    return _distmult_sc(hb, wp, pidx)


# speedup vs baseline: 1.3354x; 1.3354x over previous
import functools

import jax
import jax.numpy as jnp
from jax import lax
from jax.experimental import pallas as pl
from jax.experimental.pallas import tpu as pltpu
from jax.experimental.pallas import tpu_sc as plsc

_N = 10000
_E = 160000
_D = 256
_HD = _D // 2
_R = 4
_NC, _NS, _L = 2, 16, 16
_NW = _NC * _NS
_PW = _E // _NW
_C = 80
_NCH = -(-_PW // _C)
_TAIL = _PW - (_NCH - 1) * _C
_PAD = _NCH * _C


def _rne16(u):
    return (u + 0x7FFF + ((u >> 16) & 1)) >> 16


def _pack_pair_bf16(lo_f32, hi_f32):
    ulo = lax.bitcast_convert_type(lo_f32, jnp.int32)
    uhi = lax.bitcast_convert_type(hi_f32, jnp.int32)
    return (_rne16(ulo) & 0xFFFF) | (_rne16(uhi) << 16)


def _prep(h, w_relation, ei, et):
    bn = 1000
    rows = _E // 128 // 10

    def body(h_ref, w_ref, ei_ref, et_ref, hb_ref, wp_ref, pidx_ref):
        x = h_ref[...]
        hb_ref[...] = _pack_pair_bf16(x[:, :_HD], x[:, _HD:])
        wv = w_ref[...]
        wp_ref[...] = _pack_pair_bf16(wv[:, :_HD], wv[:, _HD:])
        pidx_ref[...] = ei_ref[0] | (ei_ref[1] << 14) | (et_ref[...] << 28)

    hb, wp, pidx = pl.pallas_call(
        body,
        grid=(10,),
        in_specs=[
            pl.BlockSpec((bn, _D), lambda i: (i, 0)),
            pl.BlockSpec((_R, _D), lambda i: (0, 0)),
            pl.BlockSpec((2, 1, rows, 128), lambda i: (0, i, 0, 0)),
            pl.BlockSpec((1, rows, 128), lambda i: (i, 0, 0)),
        ],
        out_specs=[
            pl.BlockSpec((bn, _HD), lambda i: (i, 0)),
            pl.BlockSpec((_R, _HD), lambda i: (0, 0)),
            pl.BlockSpec((1, rows, 128), lambda i: (i, 0, 0)),
        ],
        out_shape=[
            jax.ShapeDtypeStruct((_N, _HD), jnp.int32),
            jax.ShapeDtypeStruct((_R, _HD), jnp.int32),
            jax.ShapeDtypeStruct((10, rows, 128), jnp.int32),
        ],
    )(h, w_relation, ei.reshape(2, 10, rows, 128), et.reshape(10, rows, 128))
    return hb, wp.reshape(_R * _HD), pidx.reshape(_E)


_mesh = plsc.VectorSubcoreMesh(
    core_axis_name="c", subcore_axis_name="s", num_cores=_NC, num_subcores=_NS
)


@functools.partial(
    pl.kernel,
    out_type=jax.ShapeDtypeStruct((_E,), jnp.float32),
    mesh=_mesh,
    compiler_params=pltpu.CompilerParams(needs_layout_passes=False),
    scratch_types=[
        pltpu.VMEM_SHARED((_N, _HD), jnp.int32),
        pltpu.VMEM((_PAD,), jnp.int32),
        pltpu.VMEM((2, _C), jnp.int32),
        pltpu.VMEM((2, _C), jnp.int32),
        pltpu.VMEM((2, _C), jnp.int32),
        pltpu.VMEM((_R * _HD,), jnp.int32),
        pltpu.VMEM((2, _C, _HD), jnp.int32),
        pltpu.VMEM((2, _C, _HD), jnp.int32),
        pltpu.VMEM((_C,), jnp.float32),
        pltpu.SemaphoreType.DMA,
        pltpu.SemaphoreType.DMA,
        pltpu.SemaphoreType.DMA,
        pltpu.SemaphoreType.DMA,
    ],
)
def _distmult_sc(hb_hbm, wp_hbm, pidx_hbm, out_hbm,
                 tab, pidx, sidx, didx, etv, wp, s_rows, d_rows, outv,
                 sem_s0, sem_s1, sem_d0, sem_d1):
    sems = ((sem_s0, sem_d0), (sem_s1, sem_d1))
    sid = lax.axis_index("s")
    wid = sid * _NC + lax.axis_index("c")
    base = wid * _PW
    lane = lax.iota(jnp.int32, _L)
    zero16 = jnp.zeros((_L,), jnp.int32)

    toff = pl.multiple_of(sid * 624, 8)
    pltpu.sync_copy(hb_hbm.at[pl.ds(toff, 624)], tab.at[pl.ds(toff, 624)])

    @pl.when(sid == 0)
    def _tab_rem():
        pltpu.sync_copy(hb_hbm.at[pl.ds(624 * _NS, _N - 624 * _NS)],
                        tab.at[pl.ds(624 * _NS, _N - 624 * _NS)])

    pltpu.sync_copy(wp_hbm, wp)

    pltpu.sync_copy(pidx_hbm.at[pl.ds(base, _PW)], pidx.at[pl.ds(0, _PW)])
    pidx[pl.ds(_PW, _L)] = zero16
    pidx[pl.ds(_PW + _L, _L)] = zero16
    pidx[pl.ds(_PAD - _L, _L)] = zero16

    plsc.subcore_barrier()

    def _coff(j):
        return j * _C if isinstance(j, int) else pl.multiple_of(j * _C, _C)

    def _fetch(j, b):
        off = _coff(j)
        for q in range(_C // _L):
            v = pidx[pl.ds(off + q * _L, _L)]
            sidx[b, pl.ds(q * _L, _L)] = v & 0x3FFF
            didx[b, pl.ds(q * _L, _L)] = (v >> 14) & 0x3FFF
            etv[b, pl.ds(q * _L, _L)] = (v >> 28) & 0x3
        pltpu.async_copy(tab.at[sidx.at[b]], s_rows.at[b], sems[b][0])
        pltpu.async_copy(tab.at[didx.at[b]], d_rows.at[b], sems[b][1])

    def _consume(j, b, tail=False):
        pltpu.make_async_copy(tab.at[sidx.at[b]], s_rows.at[b],
                              sems[b][0]).wait()
        pltpu.make_async_copy(tab.at[didx.at[b]], d_rows.at[b],
                              sems[b][1]).wait()

        @pl.loop(0, _C // _L)
        def _group(g):
            et_vreg = etv[b, pl.ds(pl.multiple_of(g * _L, _L), _L)]
            res = jnp.zeros((_L,), jnp.float32)
            for e2 in range(_L):
                row = g * _L + e2
                et_splat = jnp.take_along_axis(
                    et_vreg, jnp.full((_L,), e2, jnp.int32), axis=0,
                    mode="promise_in_bounds")
                widx = et_splat * _HD + lane
                acc0 = jnp.zeros((_L,), jnp.float32)
                acc1 = jnp.zeros((_L,), jnp.float32)
                for k in range(_HD // _L):
                    ksl = pl.ds(k * _L, _L)
                    s32 = s_rows[b, row, ksl]
                    d32 = d_rows[b, row, ksl]
                    w32 = plsc.load_gather(wp, [widx + (k * _L)])
                    sb = plsc.bitcast(s32, jnp.bfloat16)
                    db = plsc.bitcast(d32, jnp.bfloat16)
                    wb = plsc.bitcast(w32, jnp.bfloat16)
                    p = (sb * db) * wb
                    lo, hi = plsc.unpack(p, format=plsc.PackFormat.INTERLEAVED)
                    acc0 = acc0 + lo
                    acc1 = acc1 + hi
                acc = acc0 + acc1
                for sh in (8, 4, 2, 1):
                    perm = jnp.bitwise_xor(lane, sh)
                    acc = acc + jnp.take_along_axis(
                        acc, perm, axis=0, mode="promise_in_bounds")
                res = jnp.where(lane == e2, acc, res)
            outv[pl.ds(pl.multiple_of(g * _L, _L), _L)] = res

        eoff = base + _coff(j)
        if tail:
            pltpu.sync_copy(outv.at[pl.ds(0, _TAIL)],
                            out_hbm.at[pl.ds(eoff, _TAIL)])
        else:
            pltpu.sync_copy(outv, out_hbm.at[pl.ds(eoff, _C)])

    _fetch(0, 0)

    @pl.loop(0, _NCH - 2, step=2)
    def _rounds(t):
        for b in range(2):
            _fetch(t + b + 1, 1 - b)
            _consume(t + b, b)

    _consume(_NCH - 1, 0, tail=True)


def kernel(h, edge_index, edge_type, w_relation):
    hb, wp, pidx = _prep(h, w_relation, edge_index.astype(jnp.int32),
                         edge_type.astype(jnp.int32))
    return _distmult_sc(hb, wp, pidx)
